# baseline (speedup 1.0000x reference)
import functools

import jax
import jax.numpy as jnp
from jax import lax
from jax.experimental import pallas as pl
from jax.experimental.pallas import tpu as pltpu
from jax.experimental.pallas import tpu_sc as plsc

AUDIO_LEN = 48
TOTAL_LEN = 512
IMG_LEN = TOTAL_LEN - AUDIO_LEN
D = 192
EPG = 2048
LANES = 16
SAMPLES_PER_STEP = 8


def _sc_build_adj(src_hbm, dst_hbm, u_hbm, v_hbm,
                  src_v, dst_v, u_v, v_v, sem_s, sem_d):
    wid = lax.axis_index("c") * 16 + lax.axis_index("s")

    cp_s = pltpu.make_async_copy(src_hbm.at[wid], src_v, sem_s)
    cp_d = pltpu.make_async_copy(dst_hbm.at[wid], dst_v, sem_d)
    cp_s.start()
    cp_d.start()

    zeros = jnp.zeros((LANES,), jnp.float32)

    def zero_u(r, _):
        for c in range(IMG_LEN // LANES):
            u_v[r, pl.ds(c * LANES, LANES)] = zeros
        return 0

    def zero_v(r, _):
        for c in range(AUDIO_LEN // LANES):
            v_v[r, pl.ds(c * LANES, LANES)] = zeros
        return 0

    lax.fori_loop(0, AUDIO_LEN, zero_u, 0)
    lax.fori_loop(0, IMG_LEN, zero_v, 0)

    cp_s.wait()
    cp_d.wait()

    ones = jnp.ones((LANES,), jnp.float32)

    def edge_body(i, _):
        s = src_v[pl.ds(i * LANES, LANES)]
        d = dst_v[pl.ds(i * LANES, LANES)]
        i2a = (s < IMG_LEN) & (d >= IMG_LEN)
        a2i = (s >= IMG_LEN) & (d < IMG_LEN)
        plsc.store_scatter(u_v, [jnp.where(i2a, d - IMG_LEN, 0),
                                 jnp.where(i2a, s, 0)], ones, mask=i2a)
        plsc.store_scatter(v_v, [jnp.where(a2i, d, 0),
                                 jnp.where(a2i, s - IMG_LEN, 0)], ones, mask=a2i)
        return 0

    lax.fori_loop(0, EPG // LANES, edge_body, 0)

    lane = lax.iota(jnp.int32, LANES)
    plsc.store_scatter(u_v,
                       [jnp.full((LANES,), AUDIO_LEN - 1, jnp.int32),
                        jnp.full((LANES,), IMG_LEN - 1, jnp.int32)],
                       ones, mask=lane == 0)

    cp_u = pltpu.make_async_copy(u_v, u_hbm.at[wid], sem_s)
    cp_v = pltpu.make_async_copy(v_v, v_hbm.at[wid], sem_d)
    cp_u.start()
    cp_v.start()
    cp_u.wait()
    cp_v.wait()


def _attention(z, esd, mk_adj):
    adj = mk_adj()
    ed = esd[:, 0:1]
    es = esd[:, 1:2]
    e = ed + es.T
    e = jnp.maximum(e, 0.2 * e)
    e = jnp.where(adj, e, jnp.float32(-1e9))
    p = jnp.exp(e - jnp.max(e, axis=1, keepdims=True))
    denom = jnp.sum(p, axis=1, keepdims=True)
    msg = jnp.dot(p.astype(jnp.bfloat16), z,
                  preferred_element_type=jnp.float32)
    has = jnp.any(adj, axis=1, keepdims=True)
    return jnp.where(has, msg * (1.0 / denom), 0.0)


def _ln_rows(x, g, b):
    m = jnp.mean(x, axis=1, keepdims=True)
    xc = x - m
    v = jnp.mean(xc * xc, axis=1, keepdims=True)
    return xc * lax.rsqrt(v + 1e-5) * g + b


def _han_side(h_all, n, adjs, W, a2, W_out, b_out, ln_g, ln_b):
    S = SAMPLES_PER_STEP
    z_all = jnp.dot(h_all.astype(jnp.bfloat16), W.astype(jnp.bfloat16),
                    preferred_element_type=jnp.float32).astype(jnp.bfloat16)
    Wa = jnp.dot(W, a2, preferred_element_type=jnp.float32)
    esd_all = jnp.dot(h_all, Wa, preferred_element_type=jnp.float32)
    msgs = [
        _attention(z_all[si * n:(si + 1) * n],
                   esd_all[si * n:(si + 1) * n], adjs[si])
        for si in range(S)
    ]
    msg = jnp.concatenate(msgs, axis=0)
    g = jnp.where(msg > 0.0, msg, jnp.exp(jnp.minimum(msg, 0.0)) - 1.0)
    out = jnp.dot(g.astype(jnp.bfloat16), W_out.astype(jnp.bfloat16),
                  preferred_element_type=jnp.float32) + b_out
    return _ln_rows(out, ln_g, ln_b)


def _han_kernel(bf_ref, u_ref, v_ref, wmats_ref, a2_ref, vecs_ref, out_ref):
    S = SAMPLES_PER_STEP
    w = wmats_ref[...]
    vv = vecs_ref[...]
    a2 = a2_ref[...]

    def mk_adj_i(si):
        def f():
            Ut = u_ref[si].astype(jnp.bfloat16)
            Vt = v_ref[si].astype(jnp.bfloat16)
            return jnp.dot(Vt, Ut, preferred_element_type=jnp.float32) > 0.0
        return f

    def mk_adj_a(si):
        def f():
            Ut = u_ref[si].astype(jnp.bfloat16)
            Vt = v_ref[si].astype(jnp.bfloat16)
            return jnp.dot(Ut, Vt, preferred_element_type=jnp.float32) > 0.0
        return f

    adj_i = [mk_adj_i(si) for si in range(S)]
    adj_a = [mk_adj_a(si) for si in range(S)]

    img_all = bf_ref[:, :IMG_LEN, :].reshape(S * IMG_LEN, D)
    aud_all = bf_ref[:, IMG_LEN:, :].reshape(S * AUDIO_LEN, D)

    out_i = _han_side(img_all, IMG_LEN, adj_i, w[0], a2[0], w[1],
                      vv[0:1], vv[2:3], vv[3:4])
    out_a = _han_side(aud_all, AUDIO_LEN, adj_a, w[2], a2[1], w[3],
                      vv[1:2], vv[4:5], vv[5:6])

    out_ref[:, :IMG_LEN, :] = out_i.reshape(S, IMG_LEN, D)
    out_ref[:, IMG_LEN:, :] = out_a.reshape(S, AUDIO_LEN, D)


@jax.jit
def kernel(batch_features, edge_indexes, i_params, a_params, norm1_g, norm1_b, norm2_g, norm2_b):
    Bn = batch_features.shape[0]
    src = edge_indexes[1].reshape(Bn, EPG).astype(jnp.int32)
    dst = edge_indexes[0].reshape(Bn, EPG).astype(jnp.int32)

    sc_build = functools.partial(
        pl.kernel,
        mesh=plsc.VectorSubcoreMesh(core_axis_name="c", subcore_axis_name="s"),
        out_type=[
            jax.ShapeDtypeStruct((Bn, AUDIO_LEN, IMG_LEN), jnp.float32),
            jax.ShapeDtypeStruct((Bn, IMG_LEN, AUDIO_LEN), jnp.float32),
        ],
        scratch_types=[
            pltpu.VMEM((EPG,), jnp.int32),
            pltpu.VMEM((EPG,), jnp.int32),
            pltpu.VMEM((AUDIO_LEN, IMG_LEN), jnp.float32),
            pltpu.VMEM((IMG_LEN, AUDIO_LEN), jnp.float32),
            pltpu.SemaphoreType.DMA,
            pltpu.SemaphoreType.DMA,
        ],
        compiler_params=pltpu.CompilerParams(needs_layout_passes=False),
    )(_sc_build_adj)
    u, v = sc_build(src, dst)

    wmats = jnp.stack([i_params['W'], i_params['W_out'],
                       a_params['W'], a_params['W_out']])
    a2 = jnp.stack([
        jnp.stack([i_params['a_dst'], i_params['a_src']], axis=1),
        jnp.stack([a_params['a_dst'], a_params['a_src']], axis=1),
    ])
    vecs = jnp.stack([i_params['b_out'], a_params['b_out'],
                      norm1_g, norm1_b, norm2_g, norm2_b])

    S = SAMPLES_PER_STEP
    return pl.pallas_call(
        _han_kernel,
        grid=(Bn // S,),
        in_specs=[
            pl.BlockSpec((S, TOTAL_LEN, D), lambda b: (b, 0, 0)),
            pl.BlockSpec((S, AUDIO_LEN, IMG_LEN), lambda b: (b, 0, 0)),
            pl.BlockSpec((S, IMG_LEN, AUDIO_LEN), lambda b: (b, 0, 0)),
            pl.BlockSpec((4, D, D), lambda b: (0, 0, 0)),
            pl.BlockSpec((2, D, 2), lambda b: (0, 0, 0)),
            pl.BlockSpec((6, D), lambda b: (0, 0)),
        ],
        out_specs=pl.BlockSpec((S, TOTAL_LEN, D), lambda b: (b, 0, 0)),
        out_shape=jax.ShapeDtypeStruct((Bn, TOTAL_LEN, D), jnp.float32),
        compiler_params=pltpu.CompilerParams(
            dimension_semantics=("parallel",), vmem_limit_bytes=100 * 1024 * 1024),
    )(batch_features, u, v, wmats, a2, vecs)


# speedup vs baseline: 1.0435x; 1.0435x over previous
import functools

import jax
import jax.numpy as jnp
from jax import lax
from jax.experimental import pallas as pl
from jax.experimental.pallas import tpu as pltpu
from jax.experimental.pallas import tpu_sc as plsc

AUDIO_LEN = 48
TOTAL_LEN = 512
IMG_LEN = TOTAL_LEN - AUDIO_LEN
D = 192
EPG = 2048
LANES = 16
SAMPLES_PER_STEP = 8


def _sc_build_adj(src_hbm, dst_hbm, u_hbm, v_hbm,
                  src_v, dst_v, u_v, v_v, sem_s, sem_d):
    wid = lax.axis_index("c") * 16 + lax.axis_index("s")

    cp_s = pltpu.make_async_copy(src_hbm.at[wid], src_v, sem_s)
    cp_d = pltpu.make_async_copy(dst_hbm.at[wid], dst_v, sem_d)
    cp_s.start()
    cp_d.start()

    zeros = jnp.zeros((LANES,), jnp.float32)

    def zero_u(r, _):
        for c in range(IMG_LEN // LANES):
            u_v[r, pl.ds(c * LANES, LANES)] = zeros
        return 0

    def zero_v(r, _):
        for c in range(AUDIO_LEN // LANES):
            v_v[r, pl.ds(c * LANES, LANES)] = zeros
        return 0

    lax.fori_loop(0, AUDIO_LEN, zero_u, 0)
    lax.fori_loop(0, IMG_LEN, zero_v, 0)

    cp_s.wait()
    cp_d.wait()

    ones = jnp.ones((LANES,), jnp.float32)

    def edge_body(i, _):
        s = src_v[pl.ds(i * LANES, LANES)]
        d = dst_v[pl.ds(i * LANES, LANES)]
        i2a = (s < IMG_LEN) & (d >= IMG_LEN)
        a2i = (s >= IMG_LEN) & (d < IMG_LEN)
        plsc.store_scatter(u_v, [jnp.where(i2a, d - IMG_LEN, 0),
                                 jnp.where(i2a, s, 0)], ones, mask=i2a)
        plsc.store_scatter(v_v, [jnp.where(a2i, d, 0),
                                 jnp.where(a2i, s - IMG_LEN, 0)], ones, mask=a2i)
        return 0

    lax.fori_loop(0, EPG // LANES, edge_body, 0)

    lane = lax.iota(jnp.int32, LANES)
    plsc.store_scatter(u_v,
                       [jnp.full((LANES,), AUDIO_LEN - 1, jnp.int32),
                        jnp.full((LANES,), IMG_LEN - 1, jnp.int32)],
                       ones, mask=lane == 0)

    cp_u = pltpu.make_async_copy(u_v, u_hbm.at[wid], sem_s)
    cp_v = pltpu.make_async_copy(v_v, v_hbm.at[wid], sem_d)
    cp_u.start()
    cp_v.start()
    cp_u.wait()
    cp_v.wait()


def _attention(z, esd, adj):
    ed = esd[:, 0:1]
    es = esd[:, 1:2]
    e = ed + es.T
    e = jnp.maximum(e, 0.2 * e)
    e = jnp.where(adj, e, jnp.float32(-1e9))
    p = jnp.exp(e - jnp.max(e, axis=1, keepdims=True))
    denom = jnp.sum(p, axis=1, keepdims=True)
    msg = jnp.dot(p.astype(jnp.bfloat16), z,
                  preferred_element_type=jnp.float32)
    has = jnp.any(adj, axis=1, keepdims=True)
    return jnp.where(has, msg * (1.0 / denom), 0.0)


def _ln_rows(x, g, b):
    m = jnp.mean(x, axis=1, keepdims=True)
    xc = x - m
    v = jnp.mean(xc * xc, axis=1, keepdims=True)
    return xc * lax.rsqrt(v + 1e-5) * g + b


def _han_side(h_all, n, adjs, W, a2, W_out, b_out, ln_g, ln_b):
    S = SAMPLES_PER_STEP
    z_all = jnp.dot(h_all.astype(jnp.bfloat16), W.astype(jnp.bfloat16),
                    preferred_element_type=jnp.float32).astype(jnp.bfloat16)
    Wa = jnp.dot(W, a2, preferred_element_type=jnp.float32)
    esd_all = jnp.dot(h_all, Wa, preferred_element_type=jnp.float32)
    msgs = [
        _attention(z_all[si * n:(si + 1) * n],
                   esd_all[si * n:(si + 1) * n], adjs[si])
        for si in range(S)
    ]
    msg = jnp.concatenate(msgs, axis=0)
    g = jnp.where(msg > 0.0, msg, jnp.exp(jnp.minimum(msg, 0.0)) - 1.0)
    out = jnp.dot(g.astype(jnp.bfloat16), W_out.astype(jnp.bfloat16),
                  preferred_element_type=jnp.float32) + b_out
    return _ln_rows(out, ln_g, ln_b)


def _han_kernel(bf_ref, u_ref, v_ref, wmats_ref, a2_ref, vecs_ref, out_ref):
    S = SAMPLES_PER_STEP
    w = wmats_ref[...]
    vv = vecs_ref[...]
    a2 = a2_ref[...]

    adj_i = []
    adj_a = []
    for si in range(S):
        Ut = u_ref[si].astype(jnp.bfloat16)
        Vt = v_ref[si].astype(jnp.bfloat16)
        adj_i.append(jnp.dot(Vt, Ut, preferred_element_type=jnp.float32) > 0.0)
        adj_a.append(jnp.dot(Ut, Vt, preferred_element_type=jnp.float32) > 0.0)

    img_all = bf_ref[:, :IMG_LEN, :].reshape(S * IMG_LEN, D)
    aud_all = bf_ref[:, IMG_LEN:, :].reshape(S * AUDIO_LEN, D)

    out_i = _han_side(img_all, IMG_LEN, adj_i, w[0], a2[0], w[1],
                      vv[0:1], vv[2:3], vv[3:4])
    out_a = _han_side(aud_all, AUDIO_LEN, adj_a, w[2], a2[1], w[3],
                      vv[1:2], vv[4:5], vv[5:6])

    out_ref[:, :IMG_LEN, :] = out_i.reshape(S, IMG_LEN, D)
    out_ref[:, IMG_LEN:, :] = out_a.reshape(S, AUDIO_LEN, D)


@jax.jit
def kernel(batch_features, edge_indexes, i_params, a_params, norm1_g, norm1_b, norm2_g, norm2_b):
    Bn = batch_features.shape[0]
    src = edge_indexes[1].reshape(Bn, EPG).astype(jnp.int32)
    dst = edge_indexes[0].reshape(Bn, EPG).astype(jnp.int32)

    sc_build = functools.partial(
        pl.kernel,
        mesh=plsc.VectorSubcoreMesh(core_axis_name="c", subcore_axis_name="s"),
        out_type=[
            jax.ShapeDtypeStruct((Bn, AUDIO_LEN, IMG_LEN), jnp.float32),
            jax.ShapeDtypeStruct((Bn, IMG_LEN, AUDIO_LEN), jnp.float32),
        ],
        scratch_types=[
            pltpu.VMEM((EPG,), jnp.int32),
            pltpu.VMEM((EPG,), jnp.int32),
            pltpu.VMEM((AUDIO_LEN, IMG_LEN), jnp.float32),
            pltpu.VMEM((IMG_LEN, AUDIO_LEN), jnp.float32),
            pltpu.SemaphoreType.DMA,
            pltpu.SemaphoreType.DMA,
        ],
        compiler_params=pltpu.CompilerParams(needs_layout_passes=False),
    )(_sc_build_adj)
    u, v = sc_build(src, dst)

    wmats = jnp.stack([i_params['W'], i_params['W_out'],
                       a_params['W'], a_params['W_out']])
    a2 = jnp.stack([
        jnp.stack([i_params['a_dst'], i_params['a_src']], axis=1),
        jnp.stack([a_params['a_dst'], a_params['a_src']], axis=1),
    ])
    vecs = jnp.stack([i_params['b_out'], a_params['b_out'],
                      norm1_g, norm1_b, norm2_g, norm2_b])

    S = SAMPLES_PER_STEP
    return pl.pallas_call(
        _han_kernel,
        grid=(Bn // S,),
        in_specs=[
            pl.BlockSpec((S, TOTAL_LEN, D), lambda b: (b, 0, 0)),
            pl.BlockSpec((S, AUDIO_LEN, IMG_LEN), lambda b: (b, 0, 0)),
            pl.BlockSpec((S, IMG_LEN, AUDIO_LEN), lambda b: (b, 0, 0)),
            pl.BlockSpec((4, D, D), lambda b: (0, 0, 0)),
            pl.BlockSpec((2, D, 2), lambda b: (0, 0, 0)),
            pl.BlockSpec((6, D), lambda b: (0, 0)),
        ],
        out_specs=pl.BlockSpec((S, TOTAL_LEN, D), lambda b: (b, 0, 0)),
        out_shape=jax.ShapeDtypeStruct((Bn, TOTAL_LEN, D), jnp.float32),
        compiler_params=pltpu.CompilerParams(
            dimension_semantics=("parallel",), vmem_limit_bytes=100 * 1024 * 1024),
    )(batch_features, u, v, wmats, a2, vecs)


# speedup vs baseline: 1.0516x; 1.0077x over previous
import functools

import jax
import jax.numpy as jnp
from jax import lax
from jax.experimental import pallas as pl
from jax.experimental.pallas import tpu as pltpu
from jax.experimental.pallas import tpu_sc as plsc

AUDIO_LEN = 48
TOTAL_LEN = 512
IMG_LEN = TOTAL_LEN - AUDIO_LEN
D = 192
EPG = 2048
LANES = 16
SAMPLES_PER_STEP = 4


def _sc_build_adj(src_hbm, dst_hbm, u_hbm, v_hbm,
                  src_v, dst_v, u_v, v_v, sem_s, sem_d):
    wid = lax.axis_index("c") * 16 + lax.axis_index("s")

    cp_s = pltpu.make_async_copy(src_hbm.at[wid], src_v, sem_s)
    cp_d = pltpu.make_async_copy(dst_hbm.at[wid], dst_v, sem_d)
    cp_s.start()
    cp_d.start()

    zeros = jnp.zeros((LANES,), jnp.float32)

    def zero_u(r, _):
        for c in range(IMG_LEN // LANES):
            u_v[r, pl.ds(c * LANES, LANES)] = zeros
        return 0

    def zero_v(r, _):
        for c in range(AUDIO_LEN // LANES):
            v_v[r, pl.ds(c * LANES, LANES)] = zeros
        return 0

    lax.fori_loop(0, AUDIO_LEN, zero_u, 0)
    lax.fori_loop(0, IMG_LEN, zero_v, 0)

    cp_s.wait()
    cp_d.wait()

    ones = jnp.ones((LANES,), jnp.float32)

    def edge_body(i, _):
        s = src_v[pl.ds(i * LANES, LANES)]
        d = dst_v[pl.ds(i * LANES, LANES)]
        i2a = (s < IMG_LEN) & (d >= IMG_LEN)
        a2i = (s >= IMG_LEN) & (d < IMG_LEN)
        plsc.store_scatter(u_v, [jnp.where(i2a, d - IMG_LEN, 0),
                                 jnp.where(i2a, s, 0)], ones, mask=i2a)
        plsc.store_scatter(v_v, [jnp.where(a2i, d, 0),
                                 jnp.where(a2i, s - IMG_LEN, 0)], ones, mask=a2i)
        return 0

    lax.fori_loop(0, EPG // LANES, edge_body, 0)

    lane = lax.iota(jnp.int32, LANES)
    plsc.store_scatter(u_v,
                       [jnp.full((LANES,), AUDIO_LEN - 1, jnp.int32),
                        jnp.full((LANES,), IMG_LEN - 1, jnp.int32)],
                       ones, mask=lane == 0)

    cp_u = pltpu.make_async_copy(u_v, u_hbm.at[wid], sem_s)
    cp_v = pltpu.make_async_copy(v_v, v_hbm.at[wid], sem_d)
    cp_u.start()
    cp_v.start()
    cp_u.wait()
    cp_v.wait()


def _attention(z, esd, adj):
    ed = esd[:, 0:1]
    es = esd[:, 1:2]
    e = ed + es.T
    e = jnp.maximum(e, 0.2 * e)
    e = jnp.where(adj, e, jnp.float32(-1e9))
    p = jnp.exp(e - jnp.max(e, axis=1, keepdims=True))
    denom = jnp.sum(p, axis=1, keepdims=True)
    msg = jnp.dot(p.astype(jnp.bfloat16), z,
                  preferred_element_type=jnp.float32)
    has = jnp.any(adj, axis=1, keepdims=True)
    return jnp.where(has, msg * (1.0 / denom), 0.0)


def _ln_rows(x, g, b):
    m = jnp.mean(x, axis=1, keepdims=True)
    xc = x - m
    v = jnp.mean(xc * xc, axis=1, keepdims=True)
    return xc * lax.rsqrt(v + 1e-5) * g + b


def _han_side(h_all, n, adjs, W, a2, W_out, b_out, ln_g, ln_b):
    S = SAMPLES_PER_STEP
    z_all = jnp.dot(h_all.astype(jnp.bfloat16), W.astype(jnp.bfloat16),
                    preferred_element_type=jnp.float32).astype(jnp.bfloat16)
    Wa = jnp.dot(W, a2, preferred_element_type=jnp.float32)
    esd_all = jnp.dot(h_all, Wa, preferred_element_type=jnp.float32)
    msgs = [
        _attention(z_all[si * n:(si + 1) * n],
                   esd_all[si * n:(si + 1) * n], adjs[si])
        for si in range(S)
    ]
    msg = jnp.concatenate(msgs, axis=0)
    g = jnp.where(msg > 0.0, msg, jnp.exp(jnp.minimum(msg, 0.0)) - 1.0)
    out = jnp.dot(g.astype(jnp.bfloat16), W_out.astype(jnp.bfloat16),
                  preferred_element_type=jnp.float32) + b_out
    return _ln_rows(out, ln_g, ln_b)


def _han_kernel(bf_ref, u_ref, v_ref, wmats_ref, a2_ref, vecs_ref, out_ref):
    S = SAMPLES_PER_STEP
    w = wmats_ref[...]
    vv = vecs_ref[...]
    a2 = a2_ref[...]

    adj_i = []
    adj_a = []
    for si in range(S):
        Ut = u_ref[si].astype(jnp.bfloat16)
        Vt = v_ref[si].astype(jnp.bfloat16)
        adj_i.append(jnp.dot(Vt, Ut, preferred_element_type=jnp.float32) > 0.0)
        adj_a.append(jnp.dot(Ut, Vt, preferred_element_type=jnp.float32) > 0.0)

    img_all = bf_ref[:, :IMG_LEN, :].reshape(S * IMG_LEN, D)
    aud_all = bf_ref[:, IMG_LEN:, :].reshape(S * AUDIO_LEN, D)

    out_i = _han_side(img_all, IMG_LEN, adj_i, w[0], a2[0], w[1],
                      vv[0:1], vv[2:3], vv[3:4])
    out_a = _han_side(aud_all, AUDIO_LEN, adj_a, w[2], a2[1], w[3],
                      vv[1:2], vv[4:5], vv[5:6])

    out_ref[:, :IMG_LEN, :] = out_i.reshape(S, IMG_LEN, D)
    out_ref[:, IMG_LEN:, :] = out_a.reshape(S, AUDIO_LEN, D)


@jax.jit
def kernel(batch_features, edge_indexes, i_params, a_params, norm1_g, norm1_b, norm2_g, norm2_b):
    Bn = batch_features.shape[0]
    src = edge_indexes[1].reshape(Bn, EPG).astype(jnp.int32)
    dst = edge_indexes[0].reshape(Bn, EPG).astype(jnp.int32)

    sc_build = functools.partial(
        pl.kernel,
        mesh=plsc.VectorSubcoreMesh(core_axis_name="c", subcore_axis_name="s"),
        out_type=[
            jax.ShapeDtypeStruct((Bn, AUDIO_LEN, IMG_LEN), jnp.float32),
            jax.ShapeDtypeStruct((Bn, IMG_LEN, AUDIO_LEN), jnp.float32),
        ],
        scratch_types=[
            pltpu.VMEM((EPG,), jnp.int32),
            pltpu.VMEM((EPG,), jnp.int32),
            pltpu.VMEM((AUDIO_LEN, IMG_LEN), jnp.float32),
            pltpu.VMEM((IMG_LEN, AUDIO_LEN), jnp.float32),
            pltpu.SemaphoreType.DMA,
            pltpu.SemaphoreType.DMA,
        ],
        compiler_params=pltpu.CompilerParams(needs_layout_passes=False),
    )(_sc_build_adj)
    u, v = sc_build(src, dst)

    wmats = jnp.stack([i_params['W'], i_params['W_out'],
                       a_params['W'], a_params['W_out']])
    a2 = jnp.stack([
        jnp.stack([i_params['a_dst'], i_params['a_src']], axis=1),
        jnp.stack([a_params['a_dst'], a_params['a_src']], axis=1),
    ])
    vecs = jnp.stack([i_params['b_out'], a_params['b_out'],
                      norm1_g, norm1_b, norm2_g, norm2_b])

    S = SAMPLES_PER_STEP
    return pl.pallas_call(
        _han_kernel,
        grid=(Bn // S,),
        in_specs=[
            pl.BlockSpec((S, TOTAL_LEN, D), lambda b: (b, 0, 0)),
            pl.BlockSpec((S, AUDIO_LEN, IMG_LEN), lambda b: (b, 0, 0)),
            pl.BlockSpec((S, IMG_LEN, AUDIO_LEN), lambda b: (b, 0, 0)),
            pl.BlockSpec((4, D, D), lambda b: (0, 0, 0)),
            pl.BlockSpec((2, D, 2), lambda b: (0, 0, 0)),
            pl.BlockSpec((6, D), lambda b: (0, 0)),
        ],
        out_specs=pl.BlockSpec((S, TOTAL_LEN, D), lambda b: (b, 0, 0)),
        out_shape=jax.ShapeDtypeStruct((Bn, TOTAL_LEN, D), jnp.float32),
        compiler_params=pltpu.CompilerParams(
            dimension_semantics=("parallel",), vmem_limit_bytes=100 * 1024 * 1024),
    )(batch_features, u, v, wmats, a2, vecs)
